# Initial kernel scaffold; baseline (speedup 1.0000x reference)
import functools

import jax
import jax.numpy as jnp
from jax import lax
from jax.experimental import pallas as pl
from jax.experimental.pallas import tpu as pltpu
from jax.experimental.pallas import tpu_sc as plsc

B, S, D = 2, 8192, 1024
H, DH = 16, 64
FF = 4096
CHUNK = 64
NB = 128
EPS = 1e-12
BH = B * H
NC_SC, NS_SC = 2, 16

BSA = 512
CSB = 512
NCH_B = S // CSB
BS2 = 1024
NBLK2 = S // BS2
NCHD = BS2 // CHUNK
BSF = 512
GC = 128
NGC = S // GC


def _ln(xv, g, b):
    mu = jnp.mean(xv, axis=-1, keepdims=True)
    var = jnp.mean((xv - mu) ** 2, axis=-1, keepdims=True)
    return (xv - mu) / jnp.sqrt(var + EPS) * g + b


def _a_body(x_ref, g1_ref, b1_ref, wqk_ref, wv_ref, rot_ref,
            qk_ref, v_ref, bk_ref):
    xv = x_ref[0]
    xn = _ln(xv, g1_ref[0], b1_ref[0])
    qk = jnp.dot(xn, wqk_ref[...], preferred_element_type=jnp.float32)
    v = jnp.dot(xn, wv_ref[...], preferred_element_type=jnp.float32)
    qk_ref[0] = qk
    v_ref[0] = v
    rotm = rot_ref[...]
    iota = lax.broadcasted_iota(jnp.int32, (BSA, NB // 2), 1)
    cols = []
    for h in range(H):
        r = jnp.dot(qk[:, h * DH:(h + 1) * DH], rotm,
                    preferred_element_type=jnp.float32)
        mx = jnp.max(r, axis=-1, keepdims=True)
        mn = jnp.min(r, axis=-1, keepdims=True)
        amax = jnp.min(jnp.where(r == mx, iota, NB), axis=-1, keepdims=True)
        amin = jnp.min(jnp.where(r == mn, iota, NB), axis=-1, keepdims=True)
        cols.append(jnp.where(mx >= -mn, amax, NB // 2 + amin))
    bk_ref[0] = jnp.concatenate(cols, axis=1)


def _run_a(x, ln1_g, ln1_b, Wqk, Wv, rot):
    return pl.pallas_call(
        _a_body,
        grid=(B, S // BSA),
        in_specs=[
            pl.BlockSpec((1, BSA, D), lambda b, j: (b, j, 0)),
            pl.BlockSpec((1, D), lambda b, j: (0, 0)),
            pl.BlockSpec((1, D), lambda b, j: (0, 0)),
            pl.BlockSpec((D, H * DH), lambda b, j: (0, 0)),
            pl.BlockSpec((D, H * DH), lambda b, j: (0, 0)),
            pl.BlockSpec((DH, NB // 2), lambda b, j: (0, 0)),
        ],
        out_specs=[
            pl.BlockSpec((1, BSA, H * DH), lambda b, j: (b, j, 0)),
            pl.BlockSpec((1, BSA, H * DH), lambda b, j: (b, j, 0)),
            pl.BlockSpec((1, BSA, H), lambda b, j: (b, j, 0)),
        ],
        out_shape=[
            jax.ShapeDtypeStruct((B, S, H * DH), jnp.float32),
            jax.ShapeDtypeStruct((B, S, H * DH), jnp.float32),
            jax.ShapeDtypeStruct((B, S, H), jnp.int32),
        ],
    )(x, ln1_g.reshape(1, D), ln1_b.reshape(1, D), Wqk, Wv, rot)


def _b_body(bk_ref, undo_ref, row_scr):
    bc = bk_ref[0, 0, :].reshape(S, 1)
    tri = (lax.broadcasted_iota(jnp.int32, (CSB, CSB), 0)
           >= lax.broadcasted_iota(jnp.int32, (CSB, CSB), 1)
           ).astype(jnp.float32)
    iota_l = lax.broadcasted_iota(jnp.int32, (1, NB), 1)
    run = jnp.zeros((1, NB), jnp.float32)
    for ch in range(NCH_B):
        bch = lax.slice(bc, (ch * CSB, 0), ((ch + 1) * CSB, 1))
        oh = (bch == iota_l).astype(jnp.float32)
        incl = jnp.dot(tri, oh, preferred_element_type=jnp.float32)
        rwin = jnp.sum(oh * (incl - oh + run), axis=1, keepdims=True)
        row_scr[ch * CSB:(ch + 1) * CSB] = rwin
        run = run + lax.slice(incl, (CSB - 1, 0), (CSB, NB))
    ltm = (lax.broadcasted_iota(jnp.int32, (NB, NB), 0)
           < lax.broadcasted_iota(jnp.int32, (NB, NB), 1)).astype(jnp.float32)
    base = jnp.dot(run, ltm, preferred_element_type=jnp.float32)
    for ch in range(NCH_B):
        bch = lax.slice(bc, (ch * CSB, 0), ((ch + 1) * CSB, 1))
        oh = (bch == iota_l).astype(jnp.float32)
        pick = jnp.sum(oh * base, axis=1, keepdims=True)
        row_scr[ch * CSB:(ch + 1) * CSB] = (
            row_scr[ch * CSB:(ch + 1) * CSB] + pick)
    undo_ref[0, 0, :] = row_scr[:, 0].astype(jnp.int32)


def _run_b(bkt):
    return pl.pallas_call(
        _b_body,
        grid=(BH,),
        in_specs=[pl.BlockSpec((1, 1, S), lambda r: (r, 0, 0))],
        out_specs=pl.BlockSpec((1, 1, S), lambda r: (r, 0, 0)),
        out_shape=jax.ShapeDtypeStruct((BH, 1, S), jnp.int32),
        scratch_shapes=[pltpu.VMEM((S, 1), jnp.float32)],
    )(bkt)


def _c_body(qk2d, v2d, undo_hbm, sqk_hbm, sv_hbm, g_hbm,
            undo_v, g2d, bufq, bufv, sem):
    w = lax.axis_index("s") * NC_SC + lax.axis_index("c")
    bb = w // H
    hh = w % H
    pltpu.sync_copy(undo_hbm.at[w], undo_v)

    def build(i, carry):
        u = undo_v[pl.ds(i * 16, 16)]
        tok = lax.iota(jnp.int32, 16) + i * 16
        val = (bb * S + tok) * H + hh
        plsc.store_scatter(g2d, [u >> 7, u & 127], val)
        return carry

    lax.fori_loop(0, S // 16, build, 0)
    pltpu.sync_copy(g2d, g_hbm.at[w])

    def gath(c, carry):
        pltpu.async_copy(qk2d.at[g2d.at[c]], bufq, sem).wait()
        pltpu.sync_copy(bufq, sqk_hbm.at[w, pl.ds(c * GC, GC)])
        pltpu.async_copy(v2d.at[g2d.at[c]], bufv, sem).wait()
        pltpu.sync_copy(bufv, sv_hbm.at[w, pl.ds(c * GC, GC)])
        return carry

    lax.fori_loop(0, NGC, gath, 0)


def _run_c(qk2d, v2d, undo):
    f = functools.partial(
        pl.kernel,
        out_type=(
            jax.ShapeDtypeStruct((BH, S, DH), jnp.float32),
            jax.ShapeDtypeStruct((BH, S, DH), jnp.float32),
            jax.ShapeDtypeStruct((BH, NGC, GC), jnp.int32),
        ),
        mesh=plsc.VectorSubcoreMesh(core_axis_name="c", subcore_axis_name="s"),
        scratch_types=[
            pltpu.VMEM((S,), jnp.int32),
            pltpu.VMEM((NGC, GC), jnp.int32),
            pltpu.VMEM((GC, DH), jnp.float32),
            pltpu.VMEM((GC, DH), jnp.float32),
            pltpu.SemaphoreType.DMA,
        ],
    )(_c_body)
    return f(qk2d, v2d, undo)


def _d_body(q_ref, kp_ref, v_ref, vp_ref, out_ref):
    q = q_ref[0]
    kcat = jnp.concatenate([kp_ref[0], q], axis=0)
    kcat = kcat / (jnp.sqrt(jnp.sum(kcat * kcat, axis=-1, keepdims=True))
                   + 1e-6)
    vcat = jnp.concatenate([vp_ref[0], v_ref[0]], axis=0)
    mask = (lax.broadcasted_iota(jnp.int32, (CHUNK, 2 * CHUNK), 1)
            - lax.broadcasted_iota(jnp.int32, (CHUNK, 2 * CHUNK), 0)
            ) == CHUNK
    scale = 1.0 / (DH ** 0.5)
    for i in range(NCHD):
        cq = lax.slice(q, (i * CHUNK, 0), ((i + 1) * CHUNK, DH))
        ck = lax.slice(kcat, (i * CHUNK, 0), (i * CHUNK + 2 * CHUNK, DH))
        cv = lax.slice(vcat, (i * CHUNK, 0), (i * CHUNK + 2 * CHUNK, DH))
        logits = lax.dot_general(
            cq, ck, (((1,), (1,)), ((), ())),
            preferred_element_type=jnp.float32) * scale
        logits = jnp.where(mask, -1e5, logits)
        m = jnp.max(logits, axis=-1, keepdims=True)
        e = jnp.exp(logits - m)
        attn = e / jnp.sum(e, axis=-1, keepdims=True)
        out_ref[0, i * CHUNK:(i + 1) * CHUNK, :] = jnp.dot(
            attn, cv, preferred_element_type=jnp.float32)


def _run_d(sqk, sv):
    nC = S // CHUNK
    return pl.pallas_call(
        _d_body,
        grid=(BH, NBLK2),
        in_specs=[
            pl.BlockSpec((1, BS2, DH), lambda r, j: (r, j, 0)),
            pl.BlockSpec((1, CHUNK, DH),
                         lambda r, j: (r, (j * NCHD - 1) % nC, 0)),
            pl.BlockSpec((1, BS2, DH), lambda r, j: (r, j, 0)),
            pl.BlockSpec((1, CHUNK, DH),
                         lambda r, j: (r, (j * NCHD - 1) % nC, 0)),
        ],
        out_specs=pl.BlockSpec((1, BS2, DH), lambda r, j: (r, j, 0)),
        out_shape=jax.ShapeDtypeStruct((BH, S, DH), jnp.float32),
    )(sqk, sqk, sv, sv)


def _e_body(sout_hbm, g_hbm, attn2d, g2d, buf, sem):
    w = lax.axis_index("s") * NC_SC + lax.axis_index("c")
    pltpu.sync_copy(g_hbm.at[w], g2d)

    def scat(c, carry):
        pltpu.sync_copy(sout_hbm.at[w, pl.ds(c * GC, GC)], buf)
        pltpu.async_copy(buf, attn2d.at[g2d.at[c]], sem).wait()
        return carry

    lax.fori_loop(0, NGC, scat, 0)


def _run_e(sout, g):
    f = functools.partial(
        pl.kernel,
        out_type=jax.ShapeDtypeStruct((B * S * H, DH), jnp.float32),
        mesh=plsc.VectorSubcoreMesh(core_axis_name="c", subcore_axis_name="s"),
        scratch_types=[
            pltpu.VMEM((NGC, GC), jnp.int32),
            pltpu.VMEM((GC, DH), jnp.float32),
            pltpu.SemaphoreType.DMA,
        ],
    )(_e_body)
    return f(sout, g)


def _f_body(a_ref, x_ref, wo_ref, g2_ref, b2l_ref, w1_ref, bb1_ref,
            w2_ref, bb2_ref, out_ref):
    ao = jnp.dot(a_ref[...], wo_ref[...], preferred_element_type=jnp.float32)
    x1 = ao + x_ref[...]
    xn2 = _ln(x1, g2_ref[0], b2l_ref[0])
    hmid = jnp.dot(xn2, w1_ref[...], preferred_element_type=jnp.float32) \
        + bb1_ref[0]
    hg = jax.nn.gelu(hmid)
    out_ref[...] = jnp.dot(hg, w2_ref[...],
                           preferred_element_type=jnp.float32) \
        + bb2_ref[0] + x1


def _run_f(attn, x2, Wo, ln2_g, ln2_b, W1, b1, W2, b2):
    N = B * S
    return pl.pallas_call(
        _f_body,
        grid=(N // BSF,),
        in_specs=[
            pl.BlockSpec((BSF, H * DH), lambda i: (i, 0)),
            pl.BlockSpec((BSF, D), lambda i: (i, 0)),
            pl.BlockSpec((H * DH, D), lambda i: (0, 0)),
            pl.BlockSpec((1, D), lambda i: (0, 0)),
            pl.BlockSpec((1, D), lambda i: (0, 0)),
            pl.BlockSpec((D, FF), lambda i: (0, 0)),
            pl.BlockSpec((1, FF), lambda i: (0, 0)),
            pl.BlockSpec((FF, D), lambda i: (0, 0)),
            pl.BlockSpec((1, D), lambda i: (0, 0)),
        ],
        out_specs=pl.BlockSpec((BSF, D), lambda i: (i, 0)),
        out_shape=jax.ShapeDtypeStruct((N, D), jnp.float32),
    )(attn, x2, Wo, ln2_g.reshape(1, D), ln2_b.reshape(1, D),
      W1, b1.reshape(1, FF), W2, b2.reshape(1, D))


@jax.jit
def kernel(x, ln1_g, ln1_b, ln2_g, ln2_b, Wqk, Wv, Wo, W1, b1, W2, b2, rot):
    qk, v, bkt = _run_a(x, ln1_g, ln1_b, Wqk, Wv, rot)
    bkt_t = bkt.transpose(0, 2, 1).reshape(BH, 1, S)
    undo = _run_b(bkt_t)
    sqk, sv, g = _run_c(
        qk.reshape(B * S * H, DH), v.reshape(B * S * H, DH),
        undo.reshape(BH, S))
    sout = _run_d(sqk, sv)
    attn2d = _run_e(sout, g)
    y = _run_f(attn2d.reshape(B * S, H * DH), x.reshape(B * S, D),
               Wo, ln2_g, ln2_b, W1, b1, W2, b2)
    return y.reshape(B, S, D)


# speedup vs baseline: 4.3074x; 4.3074x over previous
import functools

import jax
import jax.numpy as jnp
from jax import lax
from jax.experimental import pallas as pl
from jax.experimental.pallas import tpu as pltpu
from jax.experimental.pallas import tpu_sc as plsc

B, S, D = 2, 8192, 1024
H, DH = 16, 64
FF = 4096
CHUNK = 64
NB = 128
EPS = 1e-12
BH = B * H
PK = 2 * DH
NC_SC, NS_SC = 2, 16

BSA = 512
CSB = 512
NCH_B = S // CSB
BS2 = 1024
NBLK2 = S // BS2
NCHD = BS2 // CHUNK
BSF = 256
GC = 128
NGC = S // GC


def _ln(xv, g, b):
    mu = jnp.mean(xv, axis=-1, keepdims=True)
    var = jnp.mean((xv - mu) ** 2, axis=-1, keepdims=True)
    return (xv - mu) / jnp.sqrt(var + EPS) * g + b


def _a_body(x_ref, g1_ref, b1_ref, wqk_ref, wv_ref, rot_ref,
            comb_ref, bk_ref):
    xv = x_ref[0]
    xn = _ln(xv, g1_ref[0], b1_ref[0])
    qk = jnp.dot(xn, wqk_ref[...], preferred_element_type=jnp.float32)
    v = jnp.dot(xn, wv_ref[...], preferred_element_type=jnp.float32)
    rotm = rot_ref[...]
    iota = lax.broadcasted_iota(jnp.int32, (BSA, NB // 2), 1)
    cols = []
    for h in range(H):
        qkh = lax.slice(qk, (0, h * DH), (BSA, (h + 1) * DH))
        vh = lax.slice(v, (0, h * DH), (BSA, (h + 1) * DH))
        comb_ref[h] = jnp.concatenate([qkh, vh], axis=1)
        r = jnp.dot(qkh, rotm, preferred_element_type=jnp.float32)
        mx = jnp.max(r, axis=-1, keepdims=True)
        mn = jnp.min(r, axis=-1, keepdims=True)
        amax = jnp.min(jnp.where(r == mx, iota, NB), axis=-1, keepdims=True)
        amin = jnp.min(jnp.where(r == mn, iota, NB), axis=-1, keepdims=True)
        cols.append(jnp.where(mx >= -mn, amax, NB // 2 + amin))
    bk_ref[0] = jnp.concatenate(cols, axis=1)


def _run_a(x, ln1_g, ln1_b, Wqk, Wv, rot):
    return pl.pallas_call(
        _a_body,
        grid=(B, S // BSA),
        in_specs=[
            pl.BlockSpec((1, BSA, D), lambda b, j: (b, j, 0)),
            pl.BlockSpec((1, D), lambda b, j: (0, 0)),
            pl.BlockSpec((1, D), lambda b, j: (0, 0)),
            pl.BlockSpec((D, H * DH), lambda b, j: (0, 0)),
            pl.BlockSpec((D, H * DH), lambda b, j: (0, 0)),
            pl.BlockSpec((DH, NB // 2), lambda b, j: (0, 0)),
        ],
        out_specs=[
            pl.BlockSpec((H, BSA, PK), lambda b, j: (b, j, 0)),
            pl.BlockSpec((1, BSA, H), lambda b, j: (b, j, 0)),
        ],
        out_shape=[
            jax.ShapeDtypeStruct((BH, S, PK), jnp.float32),
            jax.ShapeDtypeStruct((B, S, H), jnp.int32),
        ],
    )(x, ln1_g.reshape(1, D), ln1_b.reshape(1, D), Wqk, Wv, rot)


def _b_body(bk_ref, undo_ref, row_scr):
    r = pl.program_id(0)
    bc = bk_ref[0, 0, :].reshape(S, 1)
    tri = (lax.broadcasted_iota(jnp.int32, (CSB, CSB), 0)
           >= lax.broadcasted_iota(jnp.int32, (CSB, CSB), 1)
           ).astype(jnp.float32)
    iota_l = lax.broadcasted_iota(jnp.int32, (1, NB), 1)
    run = jnp.zeros((1, NB), jnp.float32)
    for ch in range(NCH_B):
        bch = lax.slice(bc, (ch * CSB, 0), ((ch + 1) * CSB, 1))
        oh = (bch == iota_l).astype(jnp.float32)
        incl = jnp.dot(tri, oh, preferred_element_type=jnp.float32)
        rwin = jnp.sum(oh * (incl - oh + run), axis=1, keepdims=True)
        row_scr[ch * CSB:(ch + 1) * CSB] = rwin
        run = run + lax.slice(incl, (CSB - 1, 0), (CSB, NB))
    ltm = (lax.broadcasted_iota(jnp.int32, (NB, NB), 0)
           < lax.broadcasted_iota(jnp.int32, (NB, NB), 1)).astype(jnp.float32)
    base = jnp.dot(run, ltm, preferred_element_type=jnp.float32)
    for ch in range(NCH_B):
        bch = lax.slice(bc, (ch * CSB, 0), ((ch + 1) * CSB, 1))
        oh = (bch == iota_l).astype(jnp.float32)
        pick = jnp.sum(oh * base, axis=1, keepdims=True)
        row_scr[ch * CSB:(ch + 1) * CSB] = (
            row_scr[ch * CSB:(ch + 1) * CSB] + pick)
    undo_ref[0] = (row_scr[:, 0].astype(jnp.int32) + r * S).reshape(NGC, GC)


def _run_b(bkt):
    return pl.pallas_call(
        _b_body,
        grid=(BH,),
        in_specs=[pl.BlockSpec((1, 1, S), lambda r: (r, 0, 0))],
        out_specs=pl.BlockSpec((1, NGC, GC), lambda r: (r, 0, 0)),
        out_shape=jax.ShapeDtypeStruct((BH, NGC, GC), jnp.int32),
        scratch_shapes=[pltpu.VMEM((S, 1), jnp.float32)],
    )(bkt)


def _c_body(qkv_hbm, undo_hbm, sort_hbm, idx_v, buf, sem):
    w = lax.axis_index("s") * NC_SC + lax.axis_index("c")

    def step(c, carry):
        pltpu.sync_copy(undo_hbm.at[w, c], idx_v)
        pltpu.sync_copy(qkv_hbm.at[w, pl.ds(c * GC, GC)], buf)
        pltpu.async_copy(buf, sort_hbm.at[idx_v], sem).wait()
        return carry

    lax.fori_loop(0, NGC, step, 0)


def _run_c(qkv, undo):
    f = functools.partial(
        pl.kernel,
        out_type=jax.ShapeDtypeStruct((BH * S, PK), jnp.float32),
        mesh=plsc.VectorSubcoreMesh(core_axis_name="c", subcore_axis_name="s",
                                    num_cores=NC_SC, num_subcores=NS_SC),
        scratch_types=[
            pltpu.VMEM((GC,), jnp.int32),
            pltpu.VMEM((GC, PK), jnp.float32),
            pltpu.SemaphoreType.DMA,
        ],
    )(_c_body)
    return f(qkv, undo)


def _d_body(c_ref, cp_ref, out_ref):
    blk = c_ref[0]
    q = lax.slice(blk, (0, 0), (BS2, DH))
    kcat = jnp.concatenate(
        [lax.slice(cp_ref[0], (0, 0), (CHUNK, DH)), q], axis=0)
    kcat = kcat / (jnp.sqrt(jnp.sum(kcat * kcat, axis=-1, keepdims=True))
                   + 1e-6)
    vcat = jnp.concatenate(
        [lax.slice(cp_ref[0], (0, DH), (CHUNK, PK)),
         lax.slice(blk, (0, DH), (BS2, PK))], axis=0)
    mask = (lax.broadcasted_iota(jnp.int32, (CHUNK, 2 * CHUNK), 1)
            - lax.broadcasted_iota(jnp.int32, (CHUNK, 2 * CHUNK), 0)
            ) == CHUNK
    zpad = jnp.zeros((CHUNK, DH), jnp.float32)
    scale = 1.0 / (DH ** 0.5)
    for i in range(NCHD):
        cq = lax.slice(q, (i * CHUNK, 0), ((i + 1) * CHUNK, DH))
        ck = lax.slice(kcat, (i * CHUNK, 0), (i * CHUNK + 2 * CHUNK, DH))
        cv = lax.slice(vcat, (i * CHUNK, 0), (i * CHUNK + 2 * CHUNK, DH))
        logits = lax.dot_general(
            cq, ck, (((1,), (1,)), ((), ())),
            preferred_element_type=jnp.float32) * scale
        logits = jnp.where(mask, -1e5, logits)
        m = jnp.max(logits, axis=-1, keepdims=True)
        e = jnp.exp(logits - m)
        attn = e / jnp.sum(e, axis=-1, keepdims=True)
        o = jnp.dot(attn, cv, preferred_element_type=jnp.float32)
        out_ref[0, i * CHUNK:(i + 1) * CHUNK, :] = jnp.concatenate(
            [o, zpad], axis=1)


def _run_d(comb):
    nC = S // CHUNK
    return pl.pallas_call(
        _d_body,
        grid=(BH, NBLK2),
        in_specs=[
            pl.BlockSpec((1, BS2, PK), lambda r, j: (r, j, 0)),
            pl.BlockSpec((1, CHUNK, PK),
                         lambda r, j: (r, (j * NCHD - 1) % nC, 0)),
        ],
        out_specs=pl.BlockSpec((1, BS2, PK), lambda r, j: (r, j, 0)),
        out_shape=jax.ShapeDtypeStruct((BH, S, PK), jnp.float32),
    )(comb, comb)


def _e_body(sout_hbm, undo_hbm, attn_hbm, idx_v, buf, sem):
    w = lax.axis_index("s") * NC_SC + lax.axis_index("c")

    def step(c, carry):
        pltpu.sync_copy(undo_hbm.at[w, c], idx_v)
        pltpu.async_copy(sout_hbm.at[idx_v], buf, sem).wait()
        pltpu.sync_copy(buf, attn_hbm.at[w, pl.ds(c * GC, GC)])
        return carry

    lax.fori_loop(0, NGC, step, 0)


def _run_e(sout2d, undo):
    f = functools.partial(
        pl.kernel,
        out_type=jax.ShapeDtypeStruct((BH, S, PK), jnp.float32),
        mesh=plsc.VectorSubcoreMesh(core_axis_name="c", subcore_axis_name="s",
                                    num_cores=NC_SC, num_subcores=NS_SC),
        scratch_types=[
            pltpu.VMEM((GC,), jnp.int32),
            pltpu.VMEM((GC, PK), jnp.float32),
            pltpu.SemaphoreType.DMA,
        ],
    )(_e_body)
    return f(sout2d, undo)


def _f_body(a_ref, x_ref, wo_ref, g2_ref, b2l_ref, w1_ref, bb1_ref,
            w2_ref, bb2_ref, out_ref):
    ao = jnp.zeros((BSF, D), jnp.float32)
    for h in range(H):
        ah = lax.slice(a_ref[0, h], (0, 0), (BSF, DH))
        ao = ao + jnp.dot(ah, wo_ref[h],
                          preferred_element_type=jnp.float32)
    x1 = ao + x_ref[0]
    xn2 = _ln(x1, g2_ref[0], b2l_ref[0])
    hmid = jnp.dot(xn2, w1_ref[...], preferred_element_type=jnp.float32) \
        + bb1_ref[0]
    hg = jax.nn.gelu(hmid)
    out_ref[0] = jnp.dot(hg, w2_ref[...],
                         preferred_element_type=jnp.float32) \
        + bb2_ref[0] + x1


def _run_f(attn4d, x, Wo3, ln2_g, ln2_b, W1, b1, W2, b2):
    return pl.pallas_call(
        _f_body,
        grid=(B, S // BSF),
        in_specs=[
            pl.BlockSpec((1, H, BSF, PK), lambda b, j: (b, 0, j, 0)),
            pl.BlockSpec((1, BSF, D), lambda b, j: (b, j, 0)),
            pl.BlockSpec((H, DH, D), lambda b, j: (0, 0, 0)),
            pl.BlockSpec((1, D), lambda b, j: (0, 0)),
            pl.BlockSpec((1, D), lambda b, j: (0, 0)),
            pl.BlockSpec((D, FF), lambda b, j: (0, 0)),
            pl.BlockSpec((1, FF), lambda b, j: (0, 0)),
            pl.BlockSpec((FF, D), lambda b, j: (0, 0)),
            pl.BlockSpec((1, D), lambda b, j: (0, 0)),
        ],
        out_specs=pl.BlockSpec((1, BSF, D), lambda b, j: (b, j, 0)),
        out_shape=jax.ShapeDtypeStruct((B, S, D), jnp.float32),
    )(attn4d, x, Wo3, ln2_g.reshape(1, D), ln2_b.reshape(1, D),
      W1, b1.reshape(1, FF), W2, b2.reshape(1, D))


@jax.jit
def kernel(x, ln1_g, ln1_b, ln2_g, ln2_b, Wqk, Wv, Wo, W1, b1, W2, b2, rot):
    comb, bkt = _run_a(x, ln1_g, ln1_b, Wqk, Wv, rot)
    bkt_t = bkt.transpose(0, 2, 1).reshape(BH, 1, S)
    undo = _run_b(bkt_t)
    sqkv = _run_c(comb, undo)
    sout = _run_d(sqkv.reshape(BH, S, PK))
    attn = _run_e(sout.reshape(BH * S, PK), undo)
    y = _run_f(attn.reshape(B, H, S, PK), x, Wo.reshape(H, DH, D),
               ln2_g, ln2_b, W1, b1, W2, b2)
    return y


# speedup vs baseline: 7.0635x; 1.6398x over previous
import functools

import jax
import jax.numpy as jnp
from jax import lax
from jax.experimental import pallas as pl
from jax.experimental.pallas import tpu as pltpu
from jax.experimental.pallas import tpu_sc as plsc

B, S, D = 2, 8192, 1024
H, DH = 16, 64
FF = 4096
CHUNK = 64
NB = 128
EPS = 1e-12
BH = B * H
PK = 2 * DH
NC_SC, NS_SC = 2, 16

BSA = 512
CSB = 512
NCH_B = S // CSB
BS2 = 1024
NBLK2 = S // BS2
NCHD = BS2 // CHUNK
BSF = 256
GC = 128
NGC = S // GC


def _ln(xv, g, b):
    mu = jnp.mean(xv, axis=-1, keepdims=True)
    var = jnp.mean((xv - mu) ** 2, axis=-1, keepdims=True)
    return (xv - mu) / jnp.sqrt(var + EPS) * g + b


def _a_body(x_ref, g1_ref, b1_ref, wqk_ref, wv_ref, rot_ref,
            comb_ref, bk_ref):
    xv = x_ref[0]
    xn = _ln(xv, g1_ref[0], b1_ref[0])
    qk = jnp.dot(xn, wqk_ref[...], preferred_element_type=jnp.float32)
    v = jnp.dot(xn, wv_ref[...], preferred_element_type=jnp.float32)
    rotm = rot_ref[...]
    iota = lax.broadcasted_iota(jnp.int32, (BSA, NB // 2), 1)
    cols = []
    for h in range(H):
        qkh = lax.slice(qk, (0, h * DH), (BSA, (h + 1) * DH))
        vh = lax.slice(v, (0, h * DH), (BSA, (h + 1) * DH))
        comb_ref[h] = jnp.concatenate([qkh, vh], axis=1)
        r = jnp.dot(qkh, rotm, preferred_element_type=jnp.float32)
        mx = jnp.max(r, axis=-1, keepdims=True)
        mn = jnp.min(r, axis=-1, keepdims=True)
        amax = jnp.min(jnp.where(r == mx, iota, NB), axis=-1, keepdims=True)
        amin = jnp.min(jnp.where(r == mn, iota, NB), axis=-1, keepdims=True)
        cols.append(jnp.where(mx >= -mn, amax, NB // 2 + amin))
    bk_ref[0] = jnp.concatenate(cols, axis=1)


def _run_a(x, ln1_g, ln1_b, Wqk, Wv, rot):
    return pl.pallas_call(
        _a_body,
        grid=(B, S // BSA),
        in_specs=[
            pl.BlockSpec((1, BSA, D), lambda b, j: (b, j, 0)),
            pl.BlockSpec((1, D), lambda b, j: (0, 0)),
            pl.BlockSpec((1, D), lambda b, j: (0, 0)),
            pl.BlockSpec((D, H * DH), lambda b, j: (0, 0)),
            pl.BlockSpec((D, H * DH), lambda b, j: (0, 0)),
            pl.BlockSpec((DH, NB // 2), lambda b, j: (0, 0)),
        ],
        out_specs=[
            pl.BlockSpec((H, BSA, PK), lambda b, j: (b, j, 0)),
            pl.BlockSpec((1, BSA, H), lambda b, j: (b, j, 0)),
        ],
        out_shape=[
            jax.ShapeDtypeStruct((BH, S, PK), jnp.float32),
            jax.ShapeDtypeStruct((B, S, H), jnp.int32),
        ],
    )(x, ln1_g.reshape(1, D), ln1_b.reshape(1, D), Wqk, Wv, rot)


def _b_body(bk_ref, undo_ref, row_scr):
    r = pl.program_id(0)
    bc = bk_ref[0, 0, :].reshape(S, 1)
    tri = (lax.broadcasted_iota(jnp.int32, (CSB, CSB), 0)
           >= lax.broadcasted_iota(jnp.int32, (CSB, CSB), 1)
           ).astype(jnp.float32)
    iota_l = lax.broadcasted_iota(jnp.int32, (1, NB), 1)
    run = jnp.zeros((1, NB), jnp.float32)
    for ch in range(NCH_B):
        bch = lax.slice(bc, (ch * CSB, 0), ((ch + 1) * CSB, 1))
        oh = (bch == iota_l).astype(jnp.float32)
        incl = jnp.dot(tri, oh, preferred_element_type=jnp.float32)
        rwin = jnp.sum(oh * (incl - oh + run), axis=1, keepdims=True)
        row_scr[ch * CSB:(ch + 1) * CSB] = rwin
        run = run + lax.slice(incl, (CSB - 1, 0), (CSB, NB))
    ltm = (lax.broadcasted_iota(jnp.int32, (NB, NB), 0)
           < lax.broadcasted_iota(jnp.int32, (NB, NB), 1)).astype(jnp.float32)
    base = jnp.dot(run, ltm, preferred_element_type=jnp.float32)
    for ch in range(NCH_B):
        bch = lax.slice(bc, (ch * CSB, 0), ((ch + 1) * CSB, 1))
        oh = (bch == iota_l).astype(jnp.float32)
        pick = jnp.sum(oh * base, axis=1, keepdims=True)
        row_scr[ch * CSB:(ch + 1) * CSB] = (
            row_scr[ch * CSB:(ch + 1) * CSB] + pick)
    undo_ref[0] = (row_scr[:, 0].astype(jnp.int32) + r * S).reshape(NGC, GC)


def _run_b(bkt):
    return pl.pallas_call(
        _b_body,
        grid=(BH,),
        in_specs=[pl.BlockSpec((1, 1, S), lambda r: (r, 0, 0))],
        out_specs=pl.BlockSpec((1, NGC, GC), lambda r: (r, 0, 0)),
        out_shape=jax.ShapeDtypeStruct((BH, NGC, GC), jnp.int32),
        scratch_shapes=[pltpu.VMEM((S, 1), jnp.float32)],
    )(bkt)


def _c_body(qkv_hbm, undo_hbm, sort_hbm, idx_v, buf, sem):
    w = lax.axis_index("s") * NC_SC + lax.axis_index("c")

    def step(c, carry):
        pltpu.sync_copy(undo_hbm.at[w, c], idx_v)
        pltpu.sync_copy(qkv_hbm.at[w, pl.ds(c * GC, GC)], buf)
        pltpu.async_copy(buf, sort_hbm.at[idx_v], sem).wait()
        return carry

    lax.fori_loop(0, NGC, step, 0)


def _run_c(qkv, undo):
    f = functools.partial(
        pl.kernel,
        out_type=jax.ShapeDtypeStruct((BH * S, PK), jnp.float32),
        mesh=plsc.VectorSubcoreMesh(core_axis_name="c", subcore_axis_name="s",
                                    num_cores=NC_SC, num_subcores=NS_SC),
        scratch_types=[
            pltpu.VMEM((GC,), jnp.int32),
            pltpu.VMEM((GC, PK), jnp.float32),
            pltpu.SemaphoreType.DMA,
        ],
    )(_c_body)
    return f(qkv, undo)


def _d_body(c_ref, cp_ref, out_ref):
    blk = c_ref[0]
    q = lax.slice(blk, (0, 0), (BS2, DH))
    kcat = jnp.concatenate(
        [lax.slice(cp_ref[0], (0, 0), (CHUNK, DH)), q], axis=0)
    kcat = kcat / (jnp.sqrt(jnp.sum(kcat * kcat, axis=-1, keepdims=True))
                   + 1e-6)
    vcat = jnp.concatenate(
        [lax.slice(cp_ref[0], (0, DH), (CHUNK, PK)),
         lax.slice(blk, (0, DH), (BS2, PK))], axis=0)
    scale = 1.0 / (DH ** 0.5)
    rows = []
    for i in range(NCHD):
        cq = lax.slice(q, (i * CHUNK, 0), ((i + 1) * CHUNK, DH))
        ck = lax.slice(kcat, (i * CHUNK, 0), (i * CHUNK + 2 * CHUNK, DH))
        rows.append(lax.dot_general(
            cq, ck, (((1,), (1,)), ((), ())),
            preferred_element_type=jnp.float32))
    logits = jnp.concatenate(rows, axis=0) * scale
    mask = (lax.broadcasted_iota(jnp.int32, (BS2, 2 * CHUNK), 1)
            - (lax.broadcasted_iota(jnp.int32, (BS2, 2 * CHUNK), 0)
               & (CHUNK - 1))) == CHUNK
    logits = jnp.where(mask, -1e5, logits)
    m = jnp.max(logits, axis=-1, keepdims=True)
    e = jnp.exp(logits - m)
    attn = e / jnp.sum(e, axis=-1, keepdims=True)
    zpad = jnp.zeros((CHUNK, DH), jnp.float32)
    for i in range(NCHD):
        ai = lax.slice(attn, (i * CHUNK, 0), ((i + 1) * CHUNK, 2 * CHUNK))
        cv = lax.slice(vcat, (i * CHUNK, 0), (i * CHUNK + 2 * CHUNK, DH))
        o = jnp.dot(ai, cv, preferred_element_type=jnp.float32)
        out_ref[0, i * CHUNK:(i + 1) * CHUNK, :] = jnp.concatenate(
            [o, zpad], axis=1)


def _run_d(comb):
    nC = S // CHUNK
    return pl.pallas_call(
        _d_body,
        grid=(BH, NBLK2),
        in_specs=[
            pl.BlockSpec((1, BS2, PK), lambda r, j: (r, j, 0)),
            pl.BlockSpec((1, CHUNK, PK),
                         lambda r, j: (r, (j * NCHD - 1) % nC, 0)),
        ],
        out_specs=pl.BlockSpec((1, BS2, PK), lambda r, j: (r, j, 0)),
        out_shape=jax.ShapeDtypeStruct((BH, S, PK), jnp.float32),
    )(comb, comb)


def _e_body(sout_hbm, undo_hbm, attn_hbm, idx_v, buf, sem):
    w = lax.axis_index("s") * NC_SC + lax.axis_index("c")

    def step(c, carry):
        pltpu.sync_copy(undo_hbm.at[w, c], idx_v)
        pltpu.async_copy(sout_hbm.at[idx_v], buf, sem).wait()
        pltpu.sync_copy(buf, attn_hbm.at[w, pl.ds(c * GC, GC)])
        return carry

    lax.fori_loop(0, NGC, step, 0)


def _run_e(sout2d, undo):
    f = functools.partial(
        pl.kernel,
        out_type=jax.ShapeDtypeStruct((BH, S, PK), jnp.float32),
        mesh=plsc.VectorSubcoreMesh(core_axis_name="c", subcore_axis_name="s",
                                    num_cores=NC_SC, num_subcores=NS_SC),
        scratch_types=[
            pltpu.VMEM((GC,), jnp.int32),
            pltpu.VMEM((GC, PK), jnp.float32),
            pltpu.SemaphoreType.DMA,
        ],
    )(_e_body)
    return f(sout2d, undo)


def _f_body(a_ref, x_ref, wo_ref, g2_ref, b2l_ref, w1_ref, bb1_ref,
            w2_ref, bb2_ref, out_ref):
    ao = jnp.zeros((BSF, D), jnp.float32)
    for h in range(H):
        ah = lax.slice(a_ref[0, h], (0, 0), (BSF, DH))
        ao = ao + jnp.dot(ah, wo_ref[h],
                          preferred_element_type=jnp.float32)
    x1 = ao + x_ref[0]
    xn2 = _ln(x1, g2_ref[0], b2l_ref[0])
    hmid = jnp.dot(xn2, w1_ref[...], preferred_element_type=jnp.float32) \
        + bb1_ref[0]
    hg = jax.nn.gelu(hmid)
    out_ref[0] = jnp.dot(hg, w2_ref[...],
                         preferred_element_type=jnp.float32) \
        + bb2_ref[0] + x1


def _run_f(attn4d, x, Wo3, ln2_g, ln2_b, W1, b1, W2, b2):
    return pl.pallas_call(
        _f_body,
        grid=(B, S // BSF),
        in_specs=[
            pl.BlockSpec((1, H, BSF, PK), lambda b, j: (b, 0, j, 0)),
            pl.BlockSpec((1, BSF, D), lambda b, j: (b, j, 0)),
            pl.BlockSpec((H, DH, D), lambda b, j: (0, 0, 0)),
            pl.BlockSpec((1, D), lambda b, j: (0, 0)),
            pl.BlockSpec((1, D), lambda b, j: (0, 0)),
            pl.BlockSpec((D, FF), lambda b, j: (0, 0)),
            pl.BlockSpec((1, FF), lambda b, j: (0, 0)),
            pl.BlockSpec((FF, D), lambda b, j: (0, 0)),
            pl.BlockSpec((1, D), lambda b, j: (0, 0)),
        ],
        out_specs=pl.BlockSpec((1, BSF, D), lambda b, j: (b, j, 0)),
        out_shape=jax.ShapeDtypeStruct((B, S, D), jnp.float32),
    )(attn4d, x, Wo3, ln2_g.reshape(1, D), ln2_b.reshape(1, D),
      W1, b1.reshape(1, FF), W2, b2.reshape(1, D))


@jax.jit
def kernel(x, ln1_g, ln1_b, ln2_g, ln2_b, Wqk, Wv, Wo, W1, b1, W2, b2, rot):
    comb, bkt = _run_a(x, ln1_g, ln1_b, Wqk, Wv, rot)
    bkt_t = bkt.transpose(0, 2, 1).reshape(BH, 1, S)
    undo = _run_b(bkt_t)
    sqkv = _run_c(comb, undo)
    sout = _run_d(sqkv.reshape(BH, S, PK))
    attn = _run_e(sout.reshape(BH * S, PK), undo)
    y = _run_f(attn.reshape(B, H, S, PK), x, Wo.reshape(H, DH, D),
               ln2_g, ln2_b, W1, b1, W2, b2)
    return y
